# baseline (speedup 1.0000x reference)
import functools
import math

import jax
import jax.numpy as jnp
from jax import lax
from jax.experimental import pallas as pl
from jax.experimental.pallas import tpu as pltpu
from jax.experimental.pallas import tpu_sc as plsc

_KNN = 5
_S = 5
_SS = _S * _S
_CENTER = (_SS - 1) // 2
_SIGMA = 1.0
_CUTOFF = 1.0
_NCLS = 20
_KROWS = 32
_UNP_ROW = 25
_SROWS = 8

_NC, _NS = 2, 16
_NW = _NC * _NS
_NBUF = 3


def _inv_gauss_weights():
    x = jnp.arange(_S)
    x_grid = jnp.tile(x, _S).reshape(_S, _S)
    y_grid = x_grid.T
    mean = (_S - 1) / 2.0
    var = _SIGMA ** 2.0
    g = (1.0 / (2.0 * math.pi * var)) * jnp.exp(
        -((x_grid - mean) ** 2.0 + (y_grid - mean) ** 2.0) / (2.0 * var))
    g = g / jnp.sum(g)
    w = (1.0 - g).reshape(_SS).astype(jnp.float32)
    return jnp.concatenate([w, jnp.zeros((_KROWS - _SS,), jnp.float32)])


def _pipelined_gather(table_hbm, idx_hbm, out_hbm, bufs, ks, n_points, pt,
                      base):
    idx_v = bufs[0:_NBUF]
    buf_v = bufs[_NBUF:2 * _NBUF]
    semi = bufs[2 * _NBUF:3 * _NBUF]
    semg = bufs[3 * _NBUF:4 * _NBUF]
    sems = bufs[4 * _NBUF:5 * _NBUF]

    def row(k):
        return pl.ds(k * n_points + base, pt)

    nk = len(ks)
    ld = [None] * nk
    gat = [None] * nk
    st = [None] * nk

    def fire_st(i):
        s = i % _NBUF
        gat[i].wait()
        st[i] = pltpu.async_copy(buf_v[s], out_hbm.at[row(ks[i])], sems[s])

    ld[0] = pltpu.async_copy(idx_hbm.at[row(ks[0])], idx_v[0], semi[0])
    for i in range(nk):
        s = i % _NBUF
        ld[i].wait()
        if i >= _NBUF:
            st[i - _NBUF].wait()
        gat[i] = pltpu.async_copy(table_hbm.at[idx_v[s]], buf_v[s], semg[s])
        if i + 1 < nk:
            if i + 1 >= _NBUF:
                fire_st(i + 1 - _NBUF)
            ld[i + 1] = pltpu.async_copy(
                idx_hbm.at[row(ks[i + 1])], idx_v[(i + 1) % _NBUF],
                semi[(i + 1) % _NBUF])
    for i in range(max(0, nk - _NBUF), nk):
        fire_st(i)
        st[i].wait()


def _sc_gather_range(idx_all, rng_pad, unproj, n_points):
    pt = n_points // _NW
    mesh = plsc.VectorSubcoreMesh(core_axis_name="c", subcore_axis_name="s",
                                  num_cores=_NC, num_subcores=_NS)
    scratch = ([pltpu.VMEM((pt,), jnp.int32) for _ in range(_NBUF)]
               + [pltpu.VMEM((pt,), jnp.float32) for _ in range(_NBUF)]
               + [pltpu.SemaphoreType.DMA for _ in range(3 * _NBUF)]
               + [pltpu.VMEM((pt,), jnp.float32), pltpu.SemaphoreType.DMA])

    @functools.partial(
        pl.kernel,
        out_type=jax.ShapeDtypeStruct((_KROWS * n_points,), jnp.float32),
        mesh=mesh,
        scratch_types=scratch,
    )
    def sc1(idx_hbm, rng_hbm, unp_hbm, grng_hbm, *bufs):
        unp_v = bufs[5 * _NBUF]
        semu = bufs[5 * _NBUF + 1]
        wid = lax.axis_index("s") * _NC + lax.axis_index("c")
        base = wid * pt
        pltpu.sync_copy(unp_hbm.at[pl.ds(base, pt)], unp_v)
        u1 = pltpu.async_copy(
            unp_v, grng_hbm.at[pl.ds(_CENTER * n_points + base, pt)], semu)
        u2 = pltpu.async_copy(
            unp_v, grng_hbm.at[pl.ds(_UNP_ROW * n_points + base, pt)], semu)
        ks = [k for k in range(_SS) if k != _CENTER]
        _pipelined_gather(rng_hbm, idx_hbm, grng_hbm, bufs[:5 * _NBUF], ks,
                          n_points, pt, base)
        u1.wait()
        u2.wait()

    return sc1(idx_all, rng_pad, unproj)


def _sc_cls_vote(sel_idx, cls_pad, n_points):
    pt = n_points // _NW
    mesh = plsc.VectorSubcoreMesh(core_axis_name="c", subcore_axis_name="s",
                                  num_cores=_NC, num_subcores=_NS)
    scratch = ([pltpu.VMEM((pt,), jnp.int32) for _ in range(_KNN)]
               + [pltpu.VMEM((pt,), jnp.int32) for _ in range(_KNN)]
               + [pltpu.VMEM((pt,), jnp.int32)]
               + [pltpu.SemaphoreType.DMA for _ in range(2 * _KNN + 1)])

    @functools.partial(
        pl.kernel,
        out_type=jax.ShapeDtypeStruct((n_points,), jnp.int32),
        mesh=mesh,
        scratch_types=scratch,
        compiler_params=pltpu.CompilerParams(needs_layout_passes=False),
    )
    def sc2(selidx_hbm, cls_hbm, out_hbm, *bufs):
        idx_v = bufs[0:_KNN]
        cls_v = bufs[_KNN:2 * _KNN]
        out_v = bufs[2 * _KNN]
        semi = bufs[2 * _KNN + 1:3 * _KNN + 1]
        semg = bufs[3 * _KNN + 1:4 * _KNN + 1]
        semo = bufs[4 * _KNN + 1]
        wid = lax.axis_index("s") * _NC + lax.axis_index("c")
        base = wid * pt
        ld = [pltpu.async_copy(
            selidx_hbm.at[pl.ds(j * n_points + base, pt)], idx_v[j], semi[j])
            for j in range(_KNN)]
        gat = []
        for j in range(_KNN):
            ld[j].wait()
            gat.append(
                pltpu.async_copy(cls_hbm.at[idx_v[j]], cls_v[j], semg[j]))
        for g in gat:
            g.wait()

        def vote(gi, _):
            off = gi * 16
            sel = [cls_v[j][pl.ds(off, 16)] for j in range(_KNN)]
            ones = jnp.ones((16,), jnp.int32)
            cnt = [ones] * _KNN
            for i in range(_KNN):
                for j in range(i + 1, _KNN):
                    e = (sel[i] == sel[j]).astype(jnp.int32)
                    cnt[i] = cnt[i] + e
                    cnt[j] = cnt[j] + e
            neg = jnp.full((16,), -1000, jnp.int32)
            key = neg
            for i in range(_KNN):
                c = sel[i]
                valid = (c >= 1) & (c < _NCLS)
                key = jnp.maximum(key, jnp.where(valid, cnt[i] * 32 - c, neg))
            best = jnp.where(key == -1000, 1, 32 - (key & 31))
            out_v[pl.ds(off, 16)] = best
            return 0

        lax.fori_loop(0, pt // 16, vote, 0)
        pltpu.async_copy(out_v, out_hbm.at[pl.ds(base, pt)], semo).wait()

    return sc2(sel_idx, cls_pad)


def _tc_select_body(sentinel, grng_ref, base_ref, w_ref, offs_ref, o_ref):
    g = grng_ref[...]
    w = w_ref[...]
    offs = offs_ref[...]
    b = g.shape[1]
    base = base_ref[...].reshape(1, b)
    r = g[_UNP_ROW:_UNP_ROW + 1, :]
    rows = lax.broadcasted_iota(jnp.int32, (_KROWS, b), 0)
    d = jnp.abs(g - r) * w
    d = jnp.where(rows < _SS, d, jnp.inf)

    sel = []
    for _ in range(_KNN):
        m = jnp.min(d, axis=0, keepdims=True)
        ki = jnp.min(jnp.where(d == m, rows, _KROWS), axis=0, keepdims=True)
        hit = rows == ki
        off = jnp.max(jnp.where(hit, offs, -1), axis=0, keepdims=True)
        flat = jnp.where(m > _CUTOFF, sentinel, base + off)
        sel.append(flat)
        d = jnp.where(hit, jnp.inf, d)
    zero = jnp.zeros_like(sel[0])
    o_ref[...] = jnp.concatenate(sel + [zero] * (_SROWS - _KNN), axis=0)


def _tc_select(g_rng, base3, w_col, offs_col, sentinel, n_points, block=2048):
    nb = n_points // block
    return pl.pallas_call(
        functools.partial(_tc_select_body, sentinel),
        grid=(nb,),
        in_specs=[
            pl.BlockSpec((_KROWS, block), lambda i: (0, i)),
            pl.BlockSpec((1, 1, block), lambda i: (i, 0, 0)),
            pl.BlockSpec((_KROWS, 1), lambda i: (0, 0)),
            pl.BlockSpec((_KROWS, 1), lambda i: (0, 0)),
        ],
        out_specs=pl.BlockSpec((_SROWS, block), lambda i: (0, i)),
        out_shape=jax.ShapeDtypeStruct((_SROWS, n_points), jnp.int32),
    )(g_rng, base3, w_col, offs_col)


def _tc_vote_body(cls_ref, o_ref):
    cl = cls_ref[...]
    b = cl.shape[1]
    sel = [cl[i:i + 1, :] for i in range(_KNN)]
    ones = jnp.ones_like(sel[0])
    cnt = [ones] * _KNN
    for i in range(_KNN):
        for j in range(i + 1, _KNN):
            e = (sel[i] == sel[j]).astype(jnp.int32)
            cnt[i] = cnt[i] + e
            cnt[j] = cnt[j] + e
    neg = jnp.full_like(ones, -1000)
    key = neg
    for i in range(_KNN):
        c = sel[i]
        valid = (c >= 1) & (c < _NCLS)
        key = jnp.maximum(key, jnp.where(valid, cnt[i] * 32 - c, neg))
    best = jnp.where(key == -1000, 1, 32 - (key & 31))
    o_ref[...] = best.reshape(1, 1, b)


def _tc_vote(cls5, n_points, block=2048):
    nb = n_points // block
    return pl.pallas_call(
        _tc_vote_body,
        grid=(nb,),
        in_specs=[pl.BlockSpec((_SROWS, block), lambda i: (0, i))],
        out_specs=pl.BlockSpec((1, 1, block), lambda i: (i, 0, 0)),
        out_shape=jax.ShapeDtypeStruct((nb, 1, block), jnp.int32),
    )(cls5)


def kernel(proj_range, unproj_range, proj_argmax, px, py):
    h, w = proj_range.shape
    p = unproj_range.shape[0]
    pad = (_S - 1) // 2
    wp = w + 2 * pad
    rng_pad = jnp.pad(proj_range, pad).reshape(-1)
    npix = rng_pad.shape[0]
    cls_pad = jnp.concatenate([
        jnp.pad(proj_argmax, pad).reshape(-1),
        jnp.full((8,), _NCLS, jnp.int32)])
    sentinel = npix
    base = py * wp + px
    offs = [dy * wp + dx for dy in range(_S) for dx in range(_S)]
    offs_arr = jnp.array(offs, jnp.int32)
    w_col = _inv_gauss_weights().reshape(_KROWS, 1)
    offs_col = jnp.array(offs + [0] * (_KROWS - _SS),
                         jnp.int32).reshape(_KROWS, 1)

    nh = 4
    ph = p // nh
    outs = []
    for hh in range(nh):
        base_h = lax.slice(base, (hh * ph,), ((hh + 1) * ph,))
        unp_h = lax.slice(unproj_range, (hh * ph,), ((hh + 1) * ph,))
        idx_h = offs_arr[:, None] + base_h[None, :]
        g_rng = _sc_gather_range(idx_h.reshape(-1), rng_pad, unp_h, ph)
        g_rng = g_rng.reshape(_KROWS, ph)
        base3 = base_h.reshape(ph // 2048, 1, 2048)
        sel_idx = _tc_select(g_rng, base3, w_col, offs_col, sentinel, ph)
        outs.append(_sc_cls_vote(sel_idx.reshape(-1), cls_pad, ph))
    return jnp.concatenate(outs)


# speedup vs baseline: 1.1234x; 1.1234x over previous
import functools
import math

import jax
import jax.numpy as jnp
from jax import lax
from jax.experimental import pallas as pl
from jax.experimental.pallas import tpu as pltpu
from jax.experimental.pallas import tpu_sc as plsc

_KNN = 5
_S = 5
_SS = _S * _S
_CENTER = (_SS - 1) // 2
_SIGMA = 1.0
_CUTOFF = 1.0
_NCLS = 20
_KROWS = 32
_UNP_ROW = 25
_SROWS = 8

_NC, _NS = 2, 16
_NW = _NC * _NS
_NBUF = 3


def _inv_gauss_weights():
    x = jnp.arange(_S)
    x_grid = jnp.tile(x, _S).reshape(_S, _S)
    y_grid = x_grid.T
    mean = (_S - 1) / 2.0
    var = _SIGMA ** 2.0
    g = (1.0 / (2.0 * math.pi * var)) * jnp.exp(
        -((x_grid - mean) ** 2.0 + (y_grid - mean) ** 2.0) / (2.0 * var))
    g = g / jnp.sum(g)
    w = (1.0 - g).reshape(_SS).astype(jnp.float32)
    return jnp.concatenate([w, jnp.zeros((_KROWS - _SS,), jnp.float32)])


def _pipelined_gather(table_hbm, idx_hbm, out_hbm, bufs, ks, n_points, pt,
                      base):
    idx_v = bufs[0:_NBUF]
    buf_v = bufs[_NBUF:2 * _NBUF]
    semi = bufs[2 * _NBUF:3 * _NBUF]
    semg = bufs[3 * _NBUF:4 * _NBUF]
    sems = bufs[4 * _NBUF:5 * _NBUF]

    def row(k):
        return pl.ds(k * n_points + base, pt)

    nk = len(ks)
    ld = [None] * nk
    gat = [None] * nk
    st = [None] * nk

    def fire_st(i):
        s = i % _NBUF
        gat[i].wait()
        st[i] = pltpu.async_copy(buf_v[s], out_hbm.at[row(ks[i])], sems[s])

    ld[0] = pltpu.async_copy(idx_hbm.at[row(ks[0])], idx_v[0], semi[0])
    for i in range(nk):
        s = i % _NBUF
        ld[i].wait()
        if i >= _NBUF:
            st[i - _NBUF].wait()
        gat[i] = pltpu.async_copy(table_hbm.at[idx_v[s]], buf_v[s], semg[s])
        if i + 1 < nk:
            if i + 1 >= _NBUF:
                fire_st(i + 1 - _NBUF)
            ld[i + 1] = pltpu.async_copy(
                idx_hbm.at[row(ks[i + 1])], idx_v[(i + 1) % _NBUF],
                semi[(i + 1) % _NBUF])
    for i in range(max(0, nk - _NBUF), nk):
        fire_st(i)
        st[i].wait()


def _sc_gather_range(base_arr, rng_pad, unproj, n_points, koffs):
    pt = n_points // _NW
    mesh = plsc.VectorSubcoreMesh(core_axis_name="c", subcore_axis_name="s",
                                  num_cores=_NC, num_subcores=_NS)
    scratch = ([pltpu.VMEM((pt,), jnp.int32) for _ in range(_NBUF)]
               + [pltpu.VMEM((pt,), jnp.float32) for _ in range(_NBUF)]
               + [pltpu.VMEM((pt,), jnp.int32)]
               + [pltpu.SemaphoreType.DMA for _ in range(2 * _NBUF)]
               + [pltpu.VMEM((pt,), jnp.float32), pltpu.SemaphoreType.DMA])

    @functools.partial(
        pl.kernel,
        out_type=jax.ShapeDtypeStruct((_KROWS * n_points,), jnp.float32),
        mesh=mesh,
        scratch_types=scratch,
        compiler_params=pltpu.CompilerParams(needs_layout_passes=False),
    )
    def sc1(base_hbm, rng_hbm, unp_hbm, grng_hbm, *bufs):
        idx_v = bufs[0:_NBUF]
        buf_v = bufs[_NBUF:2 * _NBUF]
        base_v = bufs[2 * _NBUF]
        semg = bufs[2 * _NBUF + 1:3 * _NBUF + 1]
        sems = bufs[3 * _NBUF + 1:4 * _NBUF + 1]
        unp_v = bufs[4 * _NBUF + 1]
        semu = bufs[4 * _NBUF + 2]
        wid = lax.axis_index("s") * _NC + lax.axis_index("c")
        base = wid * pt
        pltpu.sync_copy(base_hbm.at[pl.ds(base, pt)], base_v)
        pltpu.sync_copy(unp_hbm.at[pl.ds(base, pt)], unp_v)
        u1 = pltpu.async_copy(
            unp_v, grng_hbm.at[pl.ds(_CENTER * n_points + base, pt)], semu)
        u2 = pltpu.async_copy(
            unp_v, grng_hbm.at[pl.ds(_UNP_ROW * n_points + base, pt)], semu)
        ks = [k for k in range(_SS) if k != _CENTER]

        def row(k):
            return pl.ds(k * n_points + base, pt)

        def calc_idx(s, off):
            def bdy(g, _):
                g16 = g * 16
                idx_v[s][pl.ds(g16, 16)] = base_v[pl.ds(g16, 16)] + off
                return 0
            lax.fori_loop(0, pt // 16, bdy, 0)

        nk = len(ks)
        gat = [None] * nk
        st = [None] * nk

        def fire_st(i):
            s = i % _NBUF
            gat[i].wait()
            st[i] = pltpu.async_copy(buf_v[s], grng_hbm.at[row(ks[i])],
                                     sems[s])

        for i in range(nk):
            s = i % _NBUF
            if i >= _NBUF:
                st[i - _NBUF].wait()
            calc_idx(s, koffs[ks[i]])
            gat[i] = pltpu.async_copy(rng_hbm.at[idx_v[s]], buf_v[s], semg[s])
            if i + 1 >= _NBUF and i + 1 < nk:
                fire_st(i + 1 - _NBUF)
        for i in range(nk - _NBUF, nk):
            fire_st(i)
            st[i].wait()
        u1.wait()
        u2.wait()

    return sc1(base_arr, rng_pad, unproj)


def _sc_cls_vote(sel_idx, cls_pad, n_points):
    pt = n_points // _NW
    mesh = plsc.VectorSubcoreMesh(core_axis_name="c", subcore_axis_name="s",
                                  num_cores=_NC, num_subcores=_NS)
    scratch = ([pltpu.VMEM((pt,), jnp.int32) for _ in range(_KNN)]
               + [pltpu.VMEM((pt,), jnp.int32) for _ in range(_KNN)]
               + [pltpu.VMEM((pt,), jnp.int32)]
               + [pltpu.SemaphoreType.DMA for _ in range(2 * _KNN + 1)])

    @functools.partial(
        pl.kernel,
        out_type=jax.ShapeDtypeStruct((n_points,), jnp.int32),
        mesh=mesh,
        scratch_types=scratch,
        compiler_params=pltpu.CompilerParams(needs_layout_passes=False),
    )
    def sc2(selidx_hbm, cls_hbm, out_hbm, *bufs):
        idx_v = bufs[0:_KNN]
        cls_v = bufs[_KNN:2 * _KNN]
        out_v = bufs[2 * _KNN]
        semi = bufs[2 * _KNN + 1:3 * _KNN + 1]
        semg = bufs[3 * _KNN + 1:4 * _KNN + 1]
        semo = bufs[4 * _KNN + 1]
        wid = lax.axis_index("s") * _NC + lax.axis_index("c")
        base = wid * pt
        ld = [pltpu.async_copy(
            selidx_hbm.at[pl.ds(j * n_points + base, pt)], idx_v[j], semi[j])
            for j in range(_KNN)]
        gat = []
        for j in range(_KNN):
            ld[j].wait()
            gat.append(
                pltpu.async_copy(cls_hbm.at[idx_v[j]], cls_v[j], semg[j]))
        for g in gat:
            g.wait()

        def vote(gi, _):
            off = gi * 16
            sel = [cls_v[j][pl.ds(off, 16)] for j in range(_KNN)]
            ones = jnp.ones((16,), jnp.int32)
            cnt = [ones] * _KNN
            for i in range(_KNN):
                for j in range(i + 1, _KNN):
                    e = (sel[i] == sel[j]).astype(jnp.int32)
                    cnt[i] = cnt[i] + e
                    cnt[j] = cnt[j] + e
            neg = jnp.full((16,), -1000, jnp.int32)
            key = neg
            for i in range(_KNN):
                c = sel[i]
                valid = (c >= 1) & (c < _NCLS)
                key = jnp.maximum(key, jnp.where(valid, cnt[i] * 32 - c, neg))
            best = jnp.where(key == -1000, 1, 32 - (key & 31))
            out_v[pl.ds(off, 16)] = best
            return 0

        lax.fori_loop(0, pt // 16, vote, 0)
        pltpu.async_copy(out_v, out_hbm.at[pl.ds(base, pt)], semo).wait()

    return sc2(sel_idx, cls_pad)


def _tc_select_body(sentinel, grng_ref, base_ref, w_ref, offs_ref, o_ref):
    g = grng_ref[...]
    w = w_ref[...]
    offs = offs_ref[...]
    b = g.shape[1]
    base = base_ref[...].reshape(1, b)
    r = g[_UNP_ROW:_UNP_ROW + 1, :]
    rows = lax.broadcasted_iota(jnp.int32, (_KROWS, b), 0)
    d = jnp.abs(g - r) * w
    d = jnp.where(rows < _SS, d, jnp.inf)

    sel = []
    for _ in range(_KNN):
        m = jnp.min(d, axis=0, keepdims=True)
        ki = jnp.min(jnp.where(d == m, rows, _KROWS), axis=0, keepdims=True)
        hit = rows == ki
        off = jnp.max(jnp.where(hit, offs, -1), axis=0, keepdims=True)
        flat = jnp.where(m > _CUTOFF, sentinel, base + off)
        sel.append(flat)
        d = jnp.where(hit, jnp.inf, d)
    zero = jnp.zeros_like(sel[0])
    o_ref[...] = jnp.concatenate(sel + [zero] * (_SROWS - _KNN), axis=0)


def _tc_select(g_rng, base3, w_col, offs_col, sentinel, n_points, block=2048):
    nb = n_points // block
    return pl.pallas_call(
        functools.partial(_tc_select_body, sentinel),
        grid=(nb,),
        in_specs=[
            pl.BlockSpec((_KROWS, block), lambda i: (0, i)),
            pl.BlockSpec((1, 1, block), lambda i: (i, 0, 0)),
            pl.BlockSpec((_KROWS, 1), lambda i: (0, 0)),
            pl.BlockSpec((_KROWS, 1), lambda i: (0, 0)),
        ],
        out_specs=pl.BlockSpec((_SROWS, block), lambda i: (0, i)),
        out_shape=jax.ShapeDtypeStruct((_SROWS, n_points), jnp.int32),
    )(g_rng, base3, w_col, offs_col)


def _tc_vote_body(cls_ref, o_ref):
    cl = cls_ref[...]
    b = cl.shape[1]
    sel = [cl[i:i + 1, :] for i in range(_KNN)]
    ones = jnp.ones_like(sel[0])
    cnt = [ones] * _KNN
    for i in range(_KNN):
        for j in range(i + 1, _KNN):
            e = (sel[i] == sel[j]).astype(jnp.int32)
            cnt[i] = cnt[i] + e
            cnt[j] = cnt[j] + e
    neg = jnp.full_like(ones, -1000)
    key = neg
    for i in range(_KNN):
        c = sel[i]
        valid = (c >= 1) & (c < _NCLS)
        key = jnp.maximum(key, jnp.where(valid, cnt[i] * 32 - c, neg))
    best = jnp.where(key == -1000, 1, 32 - (key & 31))
    o_ref[...] = best.reshape(1, 1, b)


def _tc_vote(cls5, n_points, block=2048):
    nb = n_points // block
    return pl.pallas_call(
        _tc_vote_body,
        grid=(nb,),
        in_specs=[pl.BlockSpec((_SROWS, block), lambda i: (0, i))],
        out_specs=pl.BlockSpec((1, 1, block), lambda i: (i, 0, 0)),
        out_shape=jax.ShapeDtypeStruct((nb, 1, block), jnp.int32),
    )(cls5)


def kernel(proj_range, unproj_range, proj_argmax, px, py):
    h, w = proj_range.shape
    p = unproj_range.shape[0]
    pad = (_S - 1) // 2
    wp = w + 2 * pad
    rng_pad = jnp.pad(proj_range, pad).reshape(-1)
    npix = rng_pad.shape[0]
    cls_pad = jnp.concatenate([
        jnp.pad(proj_argmax, pad).reshape(-1),
        jnp.full((8,), _NCLS, jnp.int32)])
    sentinel = npix
    base = py * wp + px
    offs = [dy * wp + dx for dy in range(_S) for dx in range(_S)]
    offs_arr = jnp.array(offs, jnp.int32)
    w_col = _inv_gauss_weights().reshape(_KROWS, 1)
    offs_col = jnp.array(offs + [0] * (_KROWS - _SS),
                         jnp.int32).reshape(_KROWS, 1)

    nh = 2
    ph = p // nh
    outs = []
    for hh in range(nh):
        base_h = lax.slice(base, (hh * ph,), ((hh + 1) * ph,))
        unp_h = lax.slice(unproj_range, (hh * ph,), ((hh + 1) * ph,))
        g_rng = _sc_gather_range(base_h, rng_pad, unp_h, ph, offs)
        g_rng = g_rng.reshape(_KROWS, ph)
        base3 = base_h.reshape(ph // 2048, 1, 2048)
        sel_idx = _tc_select(g_rng, base3, w_col, offs_col, sentinel, ph)
        outs.append(_sc_cls_vote(sel_idx.reshape(-1), cls_pad, ph))
    return jnp.concatenate(outs)


# speedup vs baseline: 1.1622x; 1.0346x over previous
import functools
import math

import jax
import jax.numpy as jnp
from jax import lax
from jax.experimental import pallas as pl
from jax.experimental.pallas import tpu as pltpu
from jax.experimental.pallas import tpu_sc as plsc

_KNN = 5
_S = 5
_SS = _S * _S
_CENTER = (_SS - 1) // 2
_SIGMA = 1.0
_CUTOFF = 1.0
_NCLS = 20
_KROWS = 32
_UNP_ROW = 25
_SROWS = 8

_NC, _NS = 2, 16
_NW = _NC * _NS
_NBUF = 3


def _inv_gauss_weights():
    x = jnp.arange(_S)
    x_grid = jnp.tile(x, _S).reshape(_S, _S)
    y_grid = x_grid.T
    mean = (_S - 1) / 2.0
    var = _SIGMA ** 2.0
    g = (1.0 / (2.0 * math.pi * var)) * jnp.exp(
        -((x_grid - mean) ** 2.0 + (y_grid - mean) ** 2.0) / (2.0 * var))
    g = g / jnp.sum(g)
    w = (1.0 - g).reshape(_SS).astype(jnp.float32)
    return jnp.concatenate([w, jnp.zeros((_KROWS - _SS,), jnp.float32)])


def _pipelined_gather(table_hbm, idx_hbm, out_hbm, bufs, ks, n_points, pt,
                      base):
    idx_v = bufs[0:_NBUF]
    buf_v = bufs[_NBUF:2 * _NBUF]
    semi = bufs[2 * _NBUF:3 * _NBUF]
    semg = bufs[3 * _NBUF:4 * _NBUF]
    sems = bufs[4 * _NBUF:5 * _NBUF]

    def row(k):
        return pl.ds(k * n_points + base, pt)

    nk = len(ks)
    ld = [None] * nk
    gat = [None] * nk
    st = [None] * nk

    def fire_st(i):
        s = i % _NBUF
        gat[i].wait()
        st[i] = pltpu.async_copy(buf_v[s], out_hbm.at[row(ks[i])], sems[s])

    ld[0] = pltpu.async_copy(idx_hbm.at[row(ks[0])], idx_v[0], semi[0])
    for i in range(nk):
        s = i % _NBUF
        ld[i].wait()
        if i >= _NBUF:
            st[i - _NBUF].wait()
        gat[i] = pltpu.async_copy(table_hbm.at[idx_v[s]], buf_v[s], semg[s])
        if i + 1 < nk:
            if i + 1 >= _NBUF:
                fire_st(i + 1 - _NBUF)
            ld[i + 1] = pltpu.async_copy(
                idx_hbm.at[row(ks[i + 1])], idx_v[(i + 1) % _NBUF],
                semi[(i + 1) % _NBUF])
    for i in range(max(0, nk - _NBUF), nk):
        fire_st(i)
        st[i].wait()


def _sc_gather_range(base_arr, rng_pad, unproj, n_points, koffs):
    pt = n_points // _NW
    mesh = plsc.VectorSubcoreMesh(core_axis_name="c", subcore_axis_name="s",
                                  num_cores=_NC, num_subcores=_NS)
    scratch = ([pltpu.VMEM((pt,), jnp.int32) for _ in range(_NBUF)]
               + [pltpu.VMEM((pt,), jnp.float32) for _ in range(_NBUF)]
               + [pltpu.VMEM((pt,), jnp.int32)]
               + [pltpu.SemaphoreType.DMA for _ in range(2 * _NBUF)]
               + [pltpu.VMEM((pt,), jnp.float32), pltpu.SemaphoreType.DMA])

    @functools.partial(
        pl.kernel,
        out_type=jax.ShapeDtypeStruct((_KROWS * n_points,), jnp.float32),
        mesh=mesh,
        scratch_types=scratch,
        compiler_params=pltpu.CompilerParams(needs_layout_passes=False),
    )
    def sc1(base_hbm, rng_hbm, unp_hbm, grng_hbm, *bufs):
        idx_v = bufs[0:_NBUF]
        buf_v = bufs[_NBUF:2 * _NBUF]
        base_v = bufs[2 * _NBUF]
        semg = bufs[2 * _NBUF + 1:3 * _NBUF + 1]
        sems = bufs[3 * _NBUF + 1:4 * _NBUF + 1]
        unp_v = bufs[4 * _NBUF + 1]
        semu = bufs[4 * _NBUF + 2]
        wid = lax.axis_index("s") * _NC + lax.axis_index("c")
        base = wid * pt
        pltpu.sync_copy(base_hbm.at[pl.ds(base, pt)], base_v)
        pltpu.sync_copy(unp_hbm.at[pl.ds(base, pt)], unp_v)
        u1 = pltpu.async_copy(
            unp_v, grng_hbm.at[pl.ds(_CENTER * n_points + base, pt)], semu)
        u2 = pltpu.async_copy(
            unp_v, grng_hbm.at[pl.ds(_UNP_ROW * n_points + base, pt)], semu)
        ks = [k for k in range(_SS) if k != _CENTER]

        def row(k):
            return pl.ds(k * n_points + base, pt)

        def calc_idx(s, off):
            def bdy(g, _):
                g16 = g * 16
                idx_v[s][pl.ds(g16, 16)] = base_v[pl.ds(g16, 16)] + off
                return 0
            lax.fori_loop(0, pt // 16, bdy, 0)

        nk = len(ks)
        gat = [None] * nk
        st = [None] * nk

        def fire_st(i):
            s = i % _NBUF
            gat[i].wait()
            st[i] = pltpu.async_copy(buf_v[s], grng_hbm.at[row(ks[i])],
                                     sems[s])

        for i in range(nk):
            s = i % _NBUF
            if i >= _NBUF:
                st[i - _NBUF].wait()
            calc_idx(s, koffs[ks[i]])
            gat[i] = pltpu.async_copy(rng_hbm.at[idx_v[s]], buf_v[s], semg[s])
            if i + 1 >= _NBUF and i + 1 < nk:
                fire_st(i + 1 - _NBUF)
        for i in range(nk - _NBUF, nk):
            fire_st(i)
            st[i].wait()
        u1.wait()
        u2.wait()

    return sc1(base_arr, rng_pad, unproj)


def _sc_cls_vote(pack_arr, base_arr, cls_pad, n_points, sentinel, wp):
    pt = n_points // _NW
    mesh = plsc.VectorSubcoreMesh(core_axis_name="c", subcore_axis_name="s",
                                  num_cores=_NC, num_subcores=_NS)
    scratch = ([pltpu.VMEM((pt,), jnp.int32) for _ in range(_KNN)]
               + [pltpu.VMEM((pt,), jnp.int32) for _ in range(_KNN)]
               + [pltpu.VMEM((pt,), jnp.int32)]
               + [pltpu.VMEM((pt,), jnp.int32)]
               + [pltpu.VMEM((pt,), jnp.int32)]
               + [pltpu.SemaphoreType.DMA for _ in range(_KNN + 1)])

    @functools.partial(
        pl.kernel,
        out_type=jax.ShapeDtypeStruct((n_points,), jnp.int32),
        mesh=mesh,
        scratch_types=scratch,
        compiler_params=pltpu.CompilerParams(needs_layout_passes=False),
    )
    def sc2(pack_hbm, basep_hbm, cls_hbm, out_hbm, *bufs):
        idx_v = bufs[0:_KNN]
        cls_v = bufs[_KNN:2 * _KNN]
        out_v = bufs[2 * _KNN]
        pack_v = bufs[2 * _KNN + 1]
        base_v = bufs[2 * _KNN + 2]
        semg = bufs[2 * _KNN + 3:3 * _KNN + 3]
        semo = bufs[3 * _KNN + 3]
        wid = lax.axis_index("s") * _NC + lax.axis_index("c")
        base = wid * pt
        pltpu.sync_copy(pack_hbm.at[pl.ds(base, pt)], pack_v)
        pltpu.sync_copy(basep_hbm.at[pl.ds(base, pt)], base_v)

        gat = []
        for j in range(_KNN):
            def decode(g, _, j=j):
                g16 = g * 16
                pk = pack_v[pl.ds(g16, 16)]
                bs = base_v[pl.ds(g16, 16)]
                k = (pk >> (5 * j)) & 31
                dy = (k * 13) >> 6
                dx = k - dy * 5
                fi = bs + dy * wp + dx
                idx_v[j][pl.ds(g16, 16)] = jnp.where(k == _SS, sentinel, fi)
                return 0
            lax.fori_loop(0, pt // 16, decode, 0)
            gat.append(
                pltpu.async_copy(cls_hbm.at[idx_v[j]], cls_v[j], semg[j]))
        for g in gat:
            g.wait()

        def vote(gi, _):
            off = gi * 16
            sel = [cls_v[j][pl.ds(off, 16)] for j in range(_KNN)]
            ones = jnp.ones((16,), jnp.int32)
            cnt = [ones] * _KNN
            for i in range(_KNN):
                for j in range(i + 1, _KNN):
                    e = (sel[i] == sel[j]).astype(jnp.int32)
                    cnt[i] = cnt[i] + e
                    cnt[j] = cnt[j] + e
            neg = jnp.full((16,), -1000, jnp.int32)
            key = neg
            for i in range(_KNN):
                c = sel[i]
                valid = (c >= 1) & (c < _NCLS)
                key = jnp.maximum(key, jnp.where(valid, cnt[i] * 32 - c, neg))
            best = jnp.where(key == -1000, 1, 32 - (key & 31))
            out_v[pl.ds(off, 16)] = best
            return 0

        lax.fori_loop(0, pt // 16, vote, 0)
        pltpu.async_copy(out_v, out_hbm.at[pl.ds(base, pt)], semo).wait()

    return sc2(pack_arr, base_arr, cls_pad)


def _tc_select_body(grng_ref, w_ref, o_ref):
    g = grng_ref[...]
    w = w_ref[...]
    b = g.shape[1]
    r = g[_UNP_ROW:_UNP_ROW + 1, :]
    rows = lax.broadcasted_iota(jnp.int32, (_KROWS, b), 0)
    d = jnp.abs(g - r) * w
    d = jnp.where(rows < _SS, d, jnp.inf)

    pack = None
    for j in range(_KNN):
        m = jnp.min(d, axis=0, keepdims=True)
        ki = jnp.min(jnp.where(d == m, rows, _KROWS), axis=0, keepdims=True)
        hit = rows == ki
        kq = jnp.where(m > _CUTOFF, _SS, ki)
        pack = kq if pack is None else pack | (kq << (5 * j))
        d = jnp.where(hit, jnp.inf, d)
    o_ref[...] = pack.reshape(1, 1, b)


def _tc_select(g_rng, w_col, n_points, block=2048):
    nb = n_points // block
    return pl.pallas_call(
        _tc_select_body,
        grid=(nb,),
        in_specs=[
            pl.BlockSpec((_KROWS, block), lambda i: (0, i)),
            pl.BlockSpec((_KROWS, 1), lambda i: (0, 0)),
        ],
        out_specs=pl.BlockSpec((1, 1, block), lambda i: (i, 0, 0)),
        out_shape=jax.ShapeDtypeStruct((nb, 1, block), jnp.int32),
    )(g_rng, w_col)


def _tc_vote_body(cls_ref, o_ref):
    cl = cls_ref[...]
    b = cl.shape[1]
    sel = [cl[i:i + 1, :] for i in range(_KNN)]
    ones = jnp.ones_like(sel[0])
    cnt = [ones] * _KNN
    for i in range(_KNN):
        for j in range(i + 1, _KNN):
            e = (sel[i] == sel[j]).astype(jnp.int32)
            cnt[i] = cnt[i] + e
            cnt[j] = cnt[j] + e
    neg = jnp.full_like(ones, -1000)
    key = neg
    for i in range(_KNN):
        c = sel[i]
        valid = (c >= 1) & (c < _NCLS)
        key = jnp.maximum(key, jnp.where(valid, cnt[i] * 32 - c, neg))
    best = jnp.where(key == -1000, 1, 32 - (key & 31))
    o_ref[...] = best.reshape(1, 1, b)


def _tc_vote(cls5, n_points, block=2048):
    nb = n_points // block
    return pl.pallas_call(
        _tc_vote_body,
        grid=(nb,),
        in_specs=[pl.BlockSpec((_SROWS, block), lambda i: (0, i))],
        out_specs=pl.BlockSpec((1, 1, block), lambda i: (i, 0, 0)),
        out_shape=jax.ShapeDtypeStruct((nb, 1, block), jnp.int32),
    )(cls5)


def kernel(proj_range, unproj_range, proj_argmax, px, py):
    h, w = proj_range.shape
    p = unproj_range.shape[0]
    pad = (_S - 1) // 2
    wp = w + 2 * pad
    rng_pad = jnp.pad(proj_range, pad).reshape(-1)
    npix = rng_pad.shape[0]
    cls_pad = jnp.concatenate([
        jnp.pad(proj_argmax, pad).reshape(-1),
        jnp.full((8,), _NCLS, jnp.int32)])
    sentinel = npix
    base = py * wp + px
    offs = [dy * wp + dx for dy in range(_S) for dx in range(_S)]
    w_col = _inv_gauss_weights().reshape(_KROWS, 1)

    nh = 2
    ph = p // nh
    outs = []
    for hh in range(nh):
        base_h = lax.slice(base, (hh * ph,), ((hh + 1) * ph,))
        unp_h = lax.slice(unproj_range, (hh * ph,), ((hh + 1) * ph,))
        g_rng = _sc_gather_range(base_h, rng_pad, unp_h, ph, offs)
        g_rng = g_rng.reshape(_KROWS, ph)
        pack = _tc_select(g_rng, w_col, ph).reshape(ph)
        outs.append(_sc_cls_vote(pack, base_h, cls_pad, ph, sentinel, wp))
    return jnp.concatenate(outs)
